# Initial kernel scaffold; baseline (speedup 1.0000x reference)
import jax
import jax.numpy as jnp
from jax.experimental import pallas as pl


def kernel(inputs_embeds, memory, attention_mask, injection_positions):
    raise NotImplementedError("write your pallas kernel here")


# speedup vs baseline: 3.2394x; 3.2394x over previous
import jax
import jax.numpy as jnp
from jax import lax
from jax.experimental import pallas as pl
from jax.experimental.pallas import tpu as pltpu
from jax.experimental.pallas import tpu_sc as plsc

B, L, D, N = 4, 2048, 4096, 16
NEW_L = L + N
NC, NS = 2, 16
NW = NC * NS
SUBS_PER_B = NW // B
ROWS_PER_W = L // SUBS_PER_B
CHUNK = 16
NCHUNK = ROWS_PER_W // CHUNK
LANES = 16


def _sc_body(emb_hbm, mem_hbm, pos_hbm, out_hbm,
             buf, idx2d, pos_v, mem_idx):
    c = lax.axis_index("c")
    s = lax.axis_index("s")
    wid = c * NS + s
    b = wid // SUBS_PER_B
    sub = wid % SUBS_PER_B

    pltpu.sync_copy(pos_hbm, pos_v)
    pos_all = pos_v[...]
    pos_vec = pos_all.at[jnp.full((LANES,), b, jnp.int32)].get(
        mode="promise_in_bounds")

    base_local = sub * ROWS_PER_W
    src_base = b * L + base_local
    out_base = b * NEW_L
    iota = lax.iota(jnp.int32, LANES)

    def idx_body(i, _):
        j = base_local + i * LANES + iota
        dst = jnp.where(j < pos_vec, j, j + N) + out_base
        idx2d[i, :] = dst
        return 0
    lax.fori_loop(0, NCHUNK, idx_body, 0)

    def copy_body(i, _):
        row0 = pl.multiple_of(src_base + i * CHUNK, CHUNK)
        pltpu.sync_copy(emb_hbm.at[pl.ds(row0, CHUNK)], buf)
        pltpu.sync_copy(buf, out_hbm.at[idx2d.at[i]])
        return 0
    lax.fori_loop(0, NCHUNK, copy_body, 0)

    @pl.when(sub == 0)
    def _():
        mem_idx[...] = out_base + pos_vec + iota
        mrow0 = pl.multiple_of(b * N, N)
        pltpu.sync_copy(mem_hbm.at[pl.ds(mrow0, N)], buf)
        pltpu.sync_copy(buf, out_hbm.at[mem_idx])


def _mask_body(am_ref, pos_ref, out_ref):
    j = lax.broadcasted_iota(jnp.int32, (B, NEW_L), 1)
    pos = pos_ref[...].reshape(B, 1)
    am = am_ref[...]
    zpad = jnp.zeros((B, N), jnp.float32)
    am_lo = jnp.concatenate([am, zpad], axis=1)
    am_hi = jnp.concatenate([zpad, am], axis=1)
    out_ref[...] = jnp.where(
        j < pos, am_lo, jnp.where(j >= pos + N, am_hi,
                                  jnp.ones((B, NEW_L), jnp.float32)))


@jax.jit
def kernel(inputs_embeds, memory, attention_mask, injection_positions):
    emb_flat = inputs_embeds.reshape(B * L, D)
    mem_flat = memory.reshape(B * N, D)
    am = attention_mask.astype(jnp.float32)
    pos32 = injection_positions.astype(jnp.int32)
    pos_pad = jnp.zeros((LANES,), jnp.int32).at[:B].set(pos32)

    mesh = plsc.VectorSubcoreMesh(core_axis_name="c", subcore_axis_name="s",
                                  num_cores=NC, num_subcores=NS)
    run = pl.kernel(
        _sc_body,
        out_type=jax.ShapeDtypeStruct((B * NEW_L, D), jnp.float32),
        mesh=mesh,
        scratch_types=[
            pltpu.VMEM((CHUNK, D), jnp.float32),
            pltpu.VMEM((NCHUNK, CHUNK), jnp.int32),
            pltpu.VMEM((LANES,), jnp.int32),
            pltpu.VMEM((LANES,), jnp.int32),
        ],
    )
    out_flat = run(emb_flat, mem_flat, pos_pad)

    new_mask = pl.pallas_call(
        _mask_body,
        out_shape=jax.ShapeDtypeStruct((B, NEW_L), jnp.float32),
    )(am, pos32)

    return out_flat.reshape(B, NEW_L, D), new_mask
